# baseline (speedup 1.0000x reference)
import functools

import jax
import jax.numpy as jnp
from jax import lax
from jax.experimental import pallas as pl
from jax.experimental.pallas import tpu as pltpu
from jax.experimental.pallas import tpu_sc as plsc

_B = 4096
_S = 200
_D = 32
_N = _B * _S
_W = 128
_ROWS = _N // _W
_NC = 2
_NS = 16
_NW = _NC * _NS
_RPW = _ROWS // _NW
_C = 8
_NCHUNK = _RPW // _C


def _gather_concat(ct, ut, cidx, uidx):
    mesh = plsc.VectorSubcoreMesh(
        core_axis_name="core", subcore_axis_name="subcore"
    )

    @functools.partial(
        pl.kernel,
        out_type=jax.ShapeDtypeStruct((_N, 2 * _D), jnp.float32),
        mesh=mesh,
        compiler_params=pltpu.CompilerParams(use_tc_tiling_on_sc=False),
        scratch_types=[
            pltpu.VMEM((2, _C, _W), jnp.int32),
            pltpu.VMEM((2, _C, _W), jnp.int32),
            pltpu.VMEM((_W, _D), jnp.float32),
            pltpu.VMEM((_W, _D), jnp.float32),
            pltpu.VMEM((_W, _D), jnp.float32),
            pltpu.VMEM((_W, _D), jnp.float32),
            pltpu.VMEM((_W, 2 * _D), jnp.float32),
            pltpu.VMEM((_W, 2 * _D), jnp.float32),
            pltpu.SemaphoreType.DMA,
            pltpu.SemaphoreType.DMA,
            pltpu.SemaphoreType.DMA,
            pltpu.SemaphoreType.DMA,
        ],
    )
    def run(ct_hbm, ut_hbm, ci_hbm, ui_hbm, o_hbm,
            ci_v, ui_v, gc0, gc1, gu0, gu1, ob0, ob1, sg0, sg1, sw0, sw1):
        wid = lax.axis_index("subcore") * _NC + lax.axis_index("core")
        row0 = wid * _RPW
        gc = (gc0, gc1)
        gu = (gu0, gu1)
        ob = (ob0, ob1)
        sg = (sg0, sg1)
        sw = (sw0, sw1)

        def load_chunk(slot, chunk):
            pltpu.sync_copy(ci_hbm.at[pl.ds(row0 + chunk * _C, _C), :],
                            ci_v.at[slot])
            pltpu.sync_copy(ui_hbm.at[pl.ds(row0 + chunk * _C, _C), :],
                            ui_v.at[slot])

        def fire(slot, jj, par):
            return (
                pltpu.async_copy(ct_hbm.at[ci_v.at[slot, jj]], gc[par],
                                 sg[par]),
                pltpu.async_copy(ut_hbm.at[ui_v.at[slot, jj]], gu[par],
                                 sg[par]),
            )

        def drain_gathers(par):
            pltpu.make_async_copy(ct_hbm.at[pl.ds(0, _W)], gc[par],
                                  sg[par]).wait()
            pltpu.make_async_copy(ut_hbm.at[pl.ds(0, _W)], gu[par],
                                  sg[par]).wait()

        def merge(par):
            src_c, src_u, dst = gc[par], gu[par], ob[par]

            @pl.loop(0, _W, step=4)
            def _(r):
                for rr in range(4):
                    for h in range(_D // 16):
                        dst.at[r + rr, pl.ds(16 * h, 16)][...] = (
                            src_c.at[r + rr, pl.ds(16 * h, 16)][...])
                        dst.at[r + rr, pl.ds(_D + 16 * h, 16)][...] = (
                            src_u.at[r + rr, pl.ds(16 * h, 16)][...])

        def drain_write(par):
            pltpu.make_async_copy(o_hbm.at[pl.ds(0, _W), :], ob[par],
                                  sw[par]).wait()

        def step(chunk, cc, jj):
            t = chunk * _C + jj
            par = jj % 2
            fire(cc, jj, par)
            drain_gathers(par)
            @pl.when(t >= 2)
            def _():
                drain_write(par)
            merge(par)
            pltpu.async_copy(
                ob[par], o_hbm.at[pl.ds((row0 + t) * _W, _W), :], sw[par])

        load_chunk(0, 0)

        @pl.loop(0, (_NCHUNK - 1) // 2)
        def _(c2):
            for cc in range(2):
                chunk = c2 * 2 + cc
                load_chunk(1 - cc, chunk + 1)
                for jj in range(_C):
                    step(chunk, cc, jj)

        for jj in range(_C):
            step(_NCHUNK - 1, 0, jj)

        drain_write(0)
        drain_write(1)

    return run(ct, ut, cidx, uidx)


def kernel(ctype, utype, ctype_table, utype_table):
    cidx = ctype.reshape(_ROWS, _W).astype(jnp.int32)
    uidx = utype.reshape(_ROWS, _W).astype(jnp.int32)
    out = _gather_concat(ctype_table, utype_table, cidx, uidx)
    return out.reshape(_B, _S, 2 * _D)


# speedup vs baseline: 1.6877x; 1.6877x over previous
import functools

import jax
import jax.numpy as jnp
from jax import lax
from jax.experimental import pallas as pl
from jax.experimental.pallas import tpu as pltpu
from jax.experimental.pallas import tpu_sc as plsc

_B = 4096
_S = 200
_D = 32
_DP = 128
_N = _B * _S
_W = 128
_ROWS = _N // _W
_NC = 2
_NS = 16
_NW = _NC * _NS
_RPW = _ROWS // _NW
_C = 8
_NCHUNK = _RPW // _C


def _gather_concat(ctu, cidx, uidx):
    mesh = plsc.VectorSubcoreMesh(
        core_axis_name="core", subcore_axis_name="subcore"
    )

    @functools.partial(
        pl.kernel,
        out_type=jax.ShapeDtypeStruct((_N, 2 * _D), jnp.float32),
        mesh=mesh,
        scratch_types=[
            pltpu.VMEM((2, _C, _W), jnp.int32),
            pltpu.VMEM((2, _C, _W), jnp.int32),
            pltpu.VMEM((_W, _DP), jnp.float32),
            pltpu.VMEM((_W, _DP), jnp.float32),
            pltpu.VMEM((_W, _DP), jnp.float32),
            pltpu.VMEM((_W, _DP), jnp.float32),
            pltpu.VMEM((_W, 2 * _D), jnp.float32),
            pltpu.VMEM((_W, 2 * _D), jnp.float32),
            pltpu.SemaphoreType.DMA,
            pltpu.SemaphoreType.DMA,
            pltpu.SemaphoreType.DMA,
            pltpu.SemaphoreType.DMA,
            pltpu.SemaphoreType.DMA,
            pltpu.SemaphoreType.DMA,
        ],
    )
    def run(ctu_hbm, ci_hbm, ui_hbm, o_hbm,
            ci_v, ui_v, gc0, gc1, gu0, gu1, ob0, ob1,
            sg0, sg1, sw0, sw1, sc0, sc1):
        wid = lax.axis_index("subcore") * _NC + lax.axis_index("core")
        row0 = wid * _RPW
        gc = (gc0, gc1)
        gu = (gu0, gu1)
        ob = (ob0, ob1)
        sg = (sg0, sg1)
        sw = (sw0, sw1)
        sc = (sc0, sc1)

        def load_chunk(slot, chunk):
            pltpu.async_copy(ci_hbm.at[pl.ds(row0 + chunk * _C, _C), :],
                             ci_v.at[slot], sc[slot])
            pltpu.async_copy(ui_hbm.at[pl.ds(row0 + chunk * _C, _C), :],
                             ui_v.at[slot], sc[slot])

        def wait_chunk(slot):
            pltpu.make_async_copy(ci_hbm.at[pl.ds(0, _C), :],
                                  ci_v.at[slot], sc[slot]).wait()
            pltpu.make_async_copy(ui_hbm.at[pl.ds(0, _C), :],
                                  ui_v.at[slot], sc[slot]).wait()

        def fire(slot, jj, par):
            return (
                pltpu.async_copy(ctu_hbm.at[ci_v.at[slot, jj]], gc[par],
                                 sg[par]),
                pltpu.async_copy(ctu_hbm.at[ui_v.at[slot, jj]], gu[par],
                                 sg[par]),
            )

        def drain_gathers(par):
            pltpu.make_async_copy(ctu_hbm.at[pl.ds(0, _W)], gc[par],
                                  sg[par]).wait()
            pltpu.make_async_copy(ctu_hbm.at[pl.ds(0, _W)], gu[par],
                                  sg[par]).wait()

        def merge(par):
            src_c, src_u, dst = gc[par], gu[par], ob[par]

            @pl.loop(0, _W, step=4)
            def _(r):
                for rr in range(4):
                    for h in range(_D // 16):
                        dst.at[r + rr, pl.ds(16 * h, 16)][...] = (
                            src_c.at[r + rr, pl.ds(16 * h, 16)][...])
                        dst.at[r + rr, pl.ds(_D + 16 * h, 16)][...] = (
                            src_u.at[r + rr, pl.ds(_D + 16 * h, 16)][...])

        def drain_write(par):
            pltpu.make_async_copy(o_hbm.at[pl.ds(0, _W), :], ob[par],
                                  sw[par]).wait()

        def step(chunk, cc, jj, last_chunk):
            t = chunk * _C + jj
            par = jj % 2
            if jj < _C - 1:
                fire(cc, jj + 1, 1 - par)
            elif not last_chunk:
                fire(1 - cc, 0, 1 - par)
            drain_gathers(par)
            @pl.when(t >= 2)
            def _():
                drain_write(par)
            merge(par)
            pltpu.async_copy(
                ob[par], o_hbm.at[pl.ds((row0 + t) * _W, _W), :], sw[par])

        load_chunk(0, 0)
        wait_chunk(0)
        load_chunk(1, 1)
        fire(0, 0, 0)

        @pl.loop(0, (_NCHUNK - 1) // 2)
        def _(c2):
            for cc in range(2):
                chunk = c2 * 2 + cc
                for jj in range(_C):
                    if jj == 1:
                        wait_chunk(1 - cc)
                    step(chunk, cc, jj, last_chunk=False)

                @pl.when(chunk + 2 < _NCHUNK)
                def _():
                    load_chunk(cc, chunk + 2)

        for jj in range(_C):
            step(_NCHUNK - 1, 0, jj, last_chunk=True)

        drain_write(0)
        drain_write(1)

    return run(ctu, cidx, uidx)


def kernel(ctype, utype, ctype_table, utype_table):
    zpad = jnp.zeros((ctype_table.shape[0], _DP - 2 * _D), jnp.float32)
    ctu = jnp.concatenate([ctype_table, utype_table, zpad], axis=1)
    cidx = ctype.reshape(_ROWS, _W).astype(jnp.int32)
    uidx = utype.reshape(_ROWS, _W).astype(jnp.int32)
    out = _gather_concat(ctu, cidx, uidx)
    return out.reshape(_B, _S, 2 * _D)


# speedup vs baseline: 1.6881x; 1.0002x over previous
import functools

import jax
import jax.numpy as jnp
from jax import lax
from jax.experimental import pallas as pl
from jax.experimental.pallas import tpu as pltpu
from jax.experimental.pallas import tpu_sc as plsc

_B = 4096
_S = 200
_D = 32
_DP = 128
_N = _B * _S
_W = 128
_ROWS = _N // _W
_NC = 2
_NS = 16
_NW = _NC * _NS
_RPW = _ROWS // _NW
_C = 8
_NCHUNK = _RPW // _C


def _gather_concat(ctu, cidx, uidx):
    mesh = plsc.VectorSubcoreMesh(
        core_axis_name="core", subcore_axis_name="subcore"
    )

    @functools.partial(
        pl.kernel,
        out_type=jax.ShapeDtypeStruct((_N, 2 * _D), jnp.float32),
        mesh=mesh,
        scratch_types=[
            pltpu.VMEM((2, _C, _W), jnp.int32),
            pltpu.VMEM((2, _C, _W), jnp.int32),
            pltpu.VMEM((_W, _DP), jnp.float32),
            pltpu.VMEM((_W, _DP), jnp.float32),
            pltpu.VMEM((_W, _DP), jnp.float32),
            pltpu.VMEM((_W, _DP), jnp.float32),
            pltpu.VMEM((_W, 2 * _D), jnp.float32),
            pltpu.VMEM((_W, 2 * _D), jnp.float32),
            pltpu.SemaphoreType.DMA,
            pltpu.SemaphoreType.DMA,
            pltpu.SemaphoreType.DMA,
            pltpu.SemaphoreType.DMA,
            pltpu.SemaphoreType.DMA,
            pltpu.SemaphoreType.DMA,
        ],
    )
    def run(ctu_hbm, ci_hbm, ui_hbm, o_hbm,
            ci_v, ui_v, gc0, gc1, gu0, gu1, ob0, ob1,
            sg0, sg1, sw0, sw1, sc0, sc1):
        wid = lax.axis_index("subcore") * _NC + lax.axis_index("core")
        row0 = wid * _RPW
        gc = (gc0, gc1)
        gu = (gu0, gu1)
        ob = (ob0, ob1)
        sg = (sg0, sg1)
        sw = (sw0, sw1)
        sc = (sc0, sc1)

        def load_chunk(slot, chunk):
            pltpu.async_copy(ci_hbm.at[pl.ds(row0 + chunk * _C, _C), :],
                             ci_v.at[slot], sc[slot])
            pltpu.async_copy(ui_hbm.at[pl.ds(row0 + chunk * _C, _C), :],
                             ui_v.at[slot], sc[slot])

        def wait_chunk(slot):
            pltpu.make_async_copy(ci_hbm.at[pl.ds(0, _C), :],
                                  ci_v.at[slot], sc[slot]).wait()
            pltpu.make_async_copy(ui_hbm.at[pl.ds(0, _C), :],
                                  ui_v.at[slot], sc[slot]).wait()

        def fire(slot, jj, par):
            h = _W // 2
            for lo in (0, h):
                pltpu.async_copy(
                    ctu_hbm.at[ci_v.at[slot, jj, pl.ds(lo, h)]],
                    gc[par].at[pl.ds(lo, h)], sg[par])
                pltpu.async_copy(
                    ctu_hbm.at[ui_v.at[slot, jj, pl.ds(lo, h)]],
                    gu[par].at[pl.ds(lo, h)], sg[par])

        def drain_gathers(par):
            h = _W // 2
            for _ in range(2):
                pltpu.make_async_copy(ctu_hbm.at[pl.ds(0, h)],
                                      gc[par].at[pl.ds(0, h)],
                                      sg[par]).wait()
                pltpu.make_async_copy(ctu_hbm.at[pl.ds(0, h)],
                                      gu[par].at[pl.ds(0, h)],
                                      sg[par]).wait()

        def merge(par):
            src_c, src_u, dst = gc[par], gu[par], ob[par]

            @pl.loop(0, _W, step=4)
            def _(r):
                for rr in range(4):
                    for h in range(_D // 16):
                        dst.at[r + rr, pl.ds(16 * h, 16)][...] = (
                            src_c.at[r + rr, pl.ds(16 * h, 16)][...])
                        dst.at[r + rr, pl.ds(_D + 16 * h, 16)][...] = (
                            src_u.at[r + rr, pl.ds(_D + 16 * h, 16)][...])

        def drain_write(par):
            pltpu.make_async_copy(o_hbm.at[pl.ds(0, _W), :], ob[par],
                                  sw[par]).wait()

        def step(chunk, cc, jj, last_chunk):
            t = chunk * _C + jj
            par = jj % 2
            if jj < _C - 1:
                fire(cc, jj + 1, 1 - par)
            elif not last_chunk:
                fire(1 - cc, 0, 1 - par)
            drain_gathers(par)
            @pl.when(t >= 2)
            def _():
                drain_write(par)
            merge(par)
            pltpu.async_copy(
                ob[par], o_hbm.at[pl.ds((row0 + t) * _W, _W), :], sw[par])

        load_chunk(0, 0)
        wait_chunk(0)
        load_chunk(1, 1)
        fire(0, 0, 0)

        @pl.loop(0, (_NCHUNK - 1) // 2)
        def _(c2):
            for cc in range(2):
                chunk = c2 * 2 + cc
                for jj in range(_C):
                    if jj == 1:
                        wait_chunk(1 - cc)
                    step(chunk, cc, jj, last_chunk=False)

                @pl.when(chunk + 2 < _NCHUNK)
                def _():
                    load_chunk(cc, chunk + 2)

        for jj in range(_C):
            step(_NCHUNK - 1, 0, jj, last_chunk=True)

        drain_write(0)
        drain_write(1)

    return run(ctu, cidx, uidx)


def kernel(ctype, utype, ctype_table, utype_table):
    zpad = jnp.zeros((ctype_table.shape[0], _DP - 2 * _D), jnp.float32)
    ctu = jnp.concatenate([ctype_table, utype_table, zpad], axis=1)
    cidx = ctype.reshape(_ROWS, _W).astype(jnp.int32)
    uidx = utype.reshape(_ROWS, _W).astype(jnp.int32)
    out = _gather_concat(ctu, cidx, uidx)
    return out.reshape(_B, _S, 2 * _D)
